# baseline (speedup 1.0000x reference)
import functools

import jax
import jax.numpy as jnp
from jax import lax
from jax.experimental import pallas as pl
from jax.experimental.pallas import tpu as pltpu
from jax.experimental.pallas import tpu_sc as plsc

N = 2_000_000
R_COLS = 3
NUM_WORKERS = 32
LANES = 16
UNROLL = 8
STEP = LANES * UNROLL
NSUB = 8
SUB = 7_808
CHUNK = NSUB * SUB
TAIL = N - NUM_WORKERS * CHUNK
TABLE_LEN = 119

R_WORDS = N * R_COLS
R_CH = 23_424
R_NCH = 8
R_PER_W = R_CH * R_NCH
R_TAIL = R_WORDS - NUM_WORKERS * R_PER_W

_mesh = plsc.VectorSubcoreMesh(core_axis_name="c", subcore_axis_name="s")


@functools.partial(
    pl.kernel,
    out_type=(
        jax.ShapeDtypeStruct((N,), jnp.int32),
        jax.ShapeDtypeStruct((R_WORDS,), jnp.float32),
    ),
    mesh=_mesh,
    compiler_params=pltpu.CompilerParams(needs_layout_passes=False),
    scratch_types=[
        pltpu.VMEM((TABLE_LEN,), jnp.int32),
        pltpu.VMEM((SUB,), jnp.int32),
        pltpu.VMEM((SUB,), jnp.int32),
        pltpu.VMEM((SUB,), jnp.int32),
        pltpu.VMEM((SUB,), jnp.int32),
        pltpu.VMEM((R_CH,), jnp.float32),
        pltpu.VMEM((R_CH,), jnp.float32),
        pltpu.VMEM((R_CH,), jnp.float32),
        pltpu.VMEM((TAIL,), jnp.int32),
        pltpu.SemaphoreType.DMA,
        pltpu.SemaphoreType.DMA,
        pltpu.SemaphoreType.DMA,
        pltpu.SemaphoreType.DMA,
        pltpu.SemaphoreType.DMA,
        pltpu.SemaphoreType.DMA,
        pltpu.SemaphoreType.DMA,
        pltpu.SemaphoreType.DMA,
        pltpu.SemaphoreType.DMA,
        pltpu.SemaphoreType.DMA,
    ],
)
def _translate(z_hbm, r_hbm, table_hbm, out_hbm, r_out_hbm,
               table_v, in_v0, in_v1, out_v0, out_v1,
               rbuf0, rbuf1, rbuf2, tail_v,
               sem_in0, sem_in1, sem_out0, sem_out1,
               sem_rin0, sem_rin1, sem_rin2,
               sem_rout0, sem_rout1, sem_rout2):
    wid = lax.axis_index("s") * 2 + lax.axis_index("c")
    base = wid * CHUNK
    r_base = wid * R_PER_W

    in_v = (in_v0, in_v1)
    out_v = (out_v0, out_v1)
    rbuf_v = (rbuf0, rbuf1, rbuf2)
    sem_in = (sem_in0, sem_in1)
    sem_out = (sem_out0, sem_out1)
    sem_rin = (sem_rin0, sem_rin1, sem_rin2)
    sem_rout = (sem_rout0, sem_rout1, sem_rout2)

    in_copies = [None] * NSUB
    out_copies = [None] * NSUB
    rin = [None] * R_NCH
    rout = [None] * R_NCH

    def start_in(s):
        b = s % 2
        in_copies[s] = pltpu.async_copy(
            z_hbm.at[pl.ds(base + s * SUB, SUB)], in_v[b], sem_in[b]
        )

    def start_rin(k):
        b = k % 3
        rin[k] = pltpu.async_copy(
            r_hbm.at[pl.ds(r_base + k * R_CH, R_CH)], rbuf_v[b],
            sem_rin[b],
        )

    def start_rout(k):
        b = k % 3
        rout[k] = pltpu.async_copy(
            rbuf_v[b], r_out_hbm.at[pl.ds(r_base + k * R_CH, R_CH)],
            sem_rout[b],
        )

    pltpu.sync_copy(table_hbm, table_v)
    start_rin(0)
    start_in(0)
    start_rin(1)
    start_rin(2)

    for s in range(NSUB):
        b = s % 2
        if s >= 2:
            rout[s - 2].wait()
            if s + 1 < R_NCH:
                start_rin(s + 1)
        if s + 1 < NSUB:
            start_in(s + 1)
        in_copies[s].wait()
        if s >= 2:
            out_copies[s - 2].wait()

        def body(i, carry, _b=b):
            off = i * STEP
            for j in range(UNROLL):
                sl = pl.ds(off + j * LANES, LANES)
                idx = in_v[_b][sl]
                out_v[_b][sl] = plsc.load_gather(table_v, [idx])
            return carry

        lax.fori_loop(0, SUB // STEP, body, 0)
        out_copies[s] = pltpu.async_copy(
            out_v[b], out_hbm.at[pl.ds(base + s * SUB, SUB)], sem_out[b]
        )
        rin[s].wait()
        start_rout(s)

    out_copies[NSUB - 2].wait()
    out_copies[NSUB - 1].wait()
    rout[R_NCH - 2].wait()
    rout[R_NCH - 1].wait()

    @pl.when(wid == 0)
    def _():
        tail_base = NUM_WORKERS * CHUNK
        pltpu.sync_copy(z_hbm.at[pl.ds(tail_base, TAIL)], tail_v)

        def tbody(i, carry):
            off = i * STEP
            for j in range(UNROLL):
                sl = pl.ds(off + j * LANES, LANES)
                idx = tail_v[sl]
                tail_v[sl] = plsc.load_gather(table_v, [idx])
            return carry

        lax.fori_loop(0, TAIL // STEP, tbody, 0)
        pltpu.sync_copy(tail_v, out_hbm.at[pl.ds(tail_base, TAIL)])

        rt_base = NUM_WORKERS * R_PER_W
        pltpu.sync_copy(r_hbm.at[pl.ds(rt_base, R_TAIL)],
                        rbuf0.at[pl.ds(0, R_TAIL)])
        pltpu.sync_copy(rbuf0.at[pl.ds(0, R_TAIL)],
                        r_out_hbm.at[pl.ds(rt_base, R_TAIL)])


def kernel(z, r, table):
    new_z, new_r = _translate(z, r.reshape(R_WORDS), table)
    return (new_z, new_r.reshape(N, R_COLS))


# speedup vs baseline: 125.3473x; 125.3473x over previous
import functools

import jax
import jax.numpy as jnp
from jax import lax
from jax.experimental import pallas as pl
from jax.experimental.pallas import tpu as pltpu
from jax.experimental.pallas import tpu_sc as plsc

N = 2_000_000
NUM_WORKERS = 32
LANES = 16
UNROLL = 8
STEP = LANES * UNROLL
NSUB = 8
SUB = 7_808
CHUNK = NSUB * SUB
TAIL = N - NUM_WORKERS * CHUNK
TABLE_LEN = 119

_mesh = plsc.VectorSubcoreMesh(core_axis_name="c", subcore_axis_name="s")


@functools.partial(
    pl.kernel,
    out_type=jax.ShapeDtypeStruct((N,), jnp.int32),
    mesh=_mesh,
    compiler_params=pltpu.CompilerParams(needs_layout_passes=False),
    scratch_types=[
        pltpu.VMEM((TABLE_LEN,), jnp.int32),
        pltpu.VMEM((SUB,), jnp.int32),
        pltpu.VMEM((SUB,), jnp.int32),
        pltpu.VMEM((SUB,), jnp.int32),
        pltpu.VMEM((SUB,), jnp.int32),
        pltpu.VMEM((TAIL,), jnp.int32),
        pltpu.SemaphoreType.DMA,
        pltpu.SemaphoreType.DMA,
        pltpu.SemaphoreType.DMA,
        pltpu.SemaphoreType.DMA,
    ],
)
def _translate(z_hbm, table_hbm, out_hbm,
               table_v, in_v0, in_v1, out_v0, out_v1, tail_v,
               sem_in0, sem_in1, sem_out0, sem_out1):
    wid = lax.axis_index("s") * 2 + lax.axis_index("c")
    base = wid * CHUNK

    in_v = (in_v0, in_v1)
    out_v = (out_v0, out_v1)
    sem_in = (sem_in0, sem_in1)
    sem_out = (sem_out0, sem_out1)

    in_copies = [None] * NSUB
    out_copies = [None] * NSUB

    def start_in(s):
        b = s % 2
        in_copies[s] = pltpu.async_copy(
            z_hbm.at[pl.ds(base + s * SUB, SUB)], in_v[b], sem_in[b]
        )

    pltpu.sync_copy(table_hbm, table_v)
    start_in(0)

    for s in range(NSUB):
        b = s % 2
        if s + 1 < NSUB:
            start_in(s + 1)
        in_copies[s].wait()
        if s >= 2:
            out_copies[s - 2].wait()

        def body(i, carry, _b=b):
            off = i * STEP
            for j in range(UNROLL):
                sl = pl.ds(off + j * LANES, LANES)
                idx = in_v[_b][sl]
                out_v[_b][sl] = plsc.load_gather(table_v, [idx])
            return carry

        lax.fori_loop(0, SUB // STEP, body, 0)
        out_copies[s] = pltpu.async_copy(
            out_v[b], out_hbm.at[pl.ds(base + s * SUB, SUB)], sem_out[b]
        )

    out_copies[NSUB - 2].wait()
    out_copies[NSUB - 1].wait()

    @pl.when(wid == 0)
    def _():
        tail_base = NUM_WORKERS * CHUNK
        pltpu.sync_copy(z_hbm.at[pl.ds(tail_base, TAIL)], tail_v)

        def tbody(i, carry):
            off = i * STEP
            for j in range(UNROLL):
                sl = pl.ds(off + j * LANES, LANES)
                idx = tail_v[sl]
                tail_v[sl] = plsc.load_gather(table_v, [idx])
            return carry

        lax.fori_loop(0, TAIL // STEP, tbody, 0)
        pltpu.sync_copy(tail_v, out_hbm.at[pl.ds(tail_base, TAIL)])


def kernel(z, r, table):
    new_z = _translate(z, table)
    return (new_z, r)
